# Initial kernel scaffold; baseline (speedup 1.0000x reference)
import jax
import jax.numpy as jnp
from jax.experimental import pallas as pl


def kernel(inputs):
    raise NotImplementedError("write your pallas kernel here")


# speedup vs baseline: 7.8967x; 7.8967x over previous
import jax
import jax.numpy as jnp
from jax import lax
from jax.experimental import pallas as pl
from jax.experimental.pallas import tpu as pltpu
from jax.experimental.pallas import tpu_sc as plsc

NC, NS, L = 2, 16, 16
NW = NC * NS
B, S, K = 4, 128, 128
ROWS = B * S * S
PAIRS = B * S
PAIRS_PER_W = PAIRS // NW
CH = K // L


def _body(in_hbm, out_hbm, idx_v, gbuf, rbuf, sem):
    wid = lax.axis_index("s") * NC + lax.axis_index("c")
    lanes = lax.iota(jnp.int32, L)

    def fetch(t, slot):
        p = wid * PAIRS_PER_W + t
        b = p // S
        i = p - b * S
        base = b * (S * S) + i
        for c in range(CH):
            idx_v[slot, pl.ds(c * L, L)] = base + S * (c * L + lanes)
        return pltpu.async_copy(in_hbm.at[idx_v.at[slot]], gbuf.at[slot], sem.at[slot])

    def drain(t, slot, dma):
        p = wid * PAIRS_PER_W + t
        dma.wait()

        def rev_row(j, carry):
            for c in range(CH):
                rbuf[j, pl.ds((CH - 1 - c) * L, L)] = lax.rev(
                    gbuf[slot, j, pl.ds(c * L, L)], (0,)
                )
            return carry

        lax.fori_loop(0, S, rev_row, 0)
        pltpu.sync_copy(rbuf, out_hbm.at[pl.ds(p * S, S)])

    dma = fetch(0, 0)
    for t in range(PAIRS_PER_W):
        nxt = fetch(t + 1, (t + 1) % 2) if t + 1 < PAIRS_PER_W else None
        drain(t, t % 2, dma)
        dma = nxt


@jax.jit
def kernel(inputs):
    rows = inputs.reshape(ROWS, K)
    out = pl.kernel(
        _body,
        out_type=jax.ShapeDtypeStruct((ROWS, K), jnp.float32),
        mesh=plsc.VectorSubcoreMesh(core_axis_name="c", subcore_axis_name="s"),
        scratch_types=[
            pltpu.VMEM((2, K), jnp.int32),
            pltpu.VMEM((2, S, K), jnp.float32),
            pltpu.VMEM((S, K), jnp.float32),
            pltpu.SemaphoreType.DMA((2,)),
        ],
    )(rows)
    return out.reshape(B, S, S, S, 1)


# speedup vs baseline: 8.3165x; 1.0532x over previous
import jax
import jax.numpy as jnp
from jax import lax
from jax.experimental import pallas as pl
from jax.experimental.pallas import tpu as pltpu
from jax.experimental.pallas import tpu_sc as plsc

NC, NS, L = 2, 16, 16
NW = NC * NS
B, S, K = 4, 128, 128
ROWS = B * S * S
PAIRS = B * S
PAIRS_PER_W = PAIRS // NW
CH = K // L


def _body(in_hbm, out_hbm, idx_v, gbuf, rbuf, sem, osem):
    wid = lax.axis_index("s") * NC + lax.axis_index("c")
    lanes = lax.iota(jnp.int32, L)

    def fetch(t, slot):
        p = wid * PAIRS_PER_W + t
        b = p // S
        i = p - b * S
        base = b * (S * S) + i
        for c in range(CH):
            idx_v[slot, pl.ds(c * L, L)] = base + S * (c * L + lanes)
        return pltpu.async_copy(in_hbm.at[idx_v.at[slot]], gbuf.at[slot], sem.at[slot])

    def drain(t, slot, dma):
        p = wid * PAIRS_PER_W + t
        dma.wait()

        def rev_row(j, carry):
            for c in range(CH):
                rbuf[slot, j, pl.ds((CH - 1 - c) * L, L)] = lax.rev(
                    gbuf[slot, j, pl.ds(c * L, L)], (0,)
                )
            return carry

        lax.fori_loop(0, S, rev_row, 0)
        return pltpu.async_copy(rbuf.at[slot], out_hbm.at[pl.ds(p * S, S)], osem.at[slot])

    dma = fetch(0, 0)
    st = [None, None]
    for t in range(PAIRS_PER_W):
        nxt = fetch(t + 1, (t + 1) % 2) if t + 1 < PAIRS_PER_W else None
        slot = t % 2
        if st[slot] is not None:
            st[slot].wait()
        st[slot] = drain(t, slot, dma)
        dma = nxt
    st[0].wait()
    st[1].wait()


@jax.jit
def kernel(inputs):
    rows = inputs.reshape(ROWS, K)
    out = pl.kernel(
        _body,
        out_type=jax.ShapeDtypeStruct((ROWS, K), jnp.float32),
        mesh=plsc.VectorSubcoreMesh(core_axis_name="c", subcore_axis_name="s"),
        scratch_types=[
            pltpu.VMEM((2, K), jnp.int32),
            pltpu.VMEM((2, S, K), jnp.float32),
            pltpu.VMEM((2, S, K), jnp.float32),
            pltpu.SemaphoreType.DMA((2,)),
            pltpu.SemaphoreType.DMA((2,)),
        ],
    )(rows)
    return out.reshape(B, S, S, S, 1)
